# baseline (speedup 1.0000x reference)
import functools

import jax
import jax.numpy as jnp
from jax import lax
from jax.experimental import pallas as pl
from jax.experimental.pallas import tpu as pltpu
from jax.experimental.pallas import tpu_sc as plsc

HIDDEN = 768
EPS = 1e-5
PAD_IDX = 0

_NUM_CORES = 2
_NUM_SUBCORES = 16

_CHUNK = 64

_LN_ROWS = 1024


def _sc_gather(table, ids):
    n = ids.shape[0]
    n_tiles = _NUM_CORES * _NUM_SUBCORES
    rows_per_tile = n // n_tiles
    n_chunks = rows_per_tile // _CHUNK
    mesh = plsc.VectorSubcoreMesh(core_axis_name="core", subcore_axis_name="subcore")

    @functools.partial(
        pl.kernel,
        out_type=jax.ShapeDtypeStruct((n, HIDDEN), jnp.float32),
        mesh=mesh,
        scratch_types=[
            pltpu.VMEM((rows_per_tile,), jnp.int32),
            pltpu.VMEM((_CHUNK, HIDDEN), jnp.float32),
            pltpu.VMEM((_CHUNK, HIDDEN), jnp.float32),
            pltpu.SemaphoreType.DMA,
            pltpu.SemaphoreType.DMA,
            pltpu.SemaphoreType.DMA,
            pltpu.SemaphoreType.DMA,
        ],
    )
    def gather_kernel(table_hbm, idx_hbm, out_hbm,
                      idx_v, buf0, buf1, g0, g1, s0, s1):
        wid = lax.axis_index("subcore") * _NUM_CORES + lax.axis_index("core")
        base = wid * rows_per_tile
        pltpu.sync_copy(idx_hbm.at[pl.ds(base, rows_per_tile)], idx_v)

        def gather_copy(c, buf, sem):
            return pltpu.make_async_copy(
                table_hbm.at[idx_v.at[pl.ds(c * _CHUNK, _CHUNK)]], buf, sem
            )

        def store_copy(c, buf, sem):
            return pltpu.make_async_copy(
                buf, out_hbm.at[pl.ds(base + c * _CHUNK, _CHUNK)], sem
            )

        gather_copy(0, buf0, g0).start()

        @pl.loop(0, n_chunks, step=2)
        def _(c):
            @pl.when(c + 1 < n_chunks)
            def _():
                gather_copy(c + 1, buf1, g1).start()

            gather_copy(c, buf0, g0).wait()
            store_copy(c, buf0, s0).start()
            store_copy(c, buf0, s0).wait()

            @pl.when(c + 2 < n_chunks)
            def _():
                gather_copy(c + 2, buf0, g0).start()

            @pl.when(c + 1 < n_chunks)
            def _():
                gather_copy(c + 1, buf1, g1).wait()
                store_copy(c + 1, buf1, s1).start()
                store_copy(c + 1, buf1, s1).wait()

    return gather_kernel(table, ids)


_K_SLICES = 4


def _tc_layernorm_slice(rows, ids_col, gamma_row, beta_row, big, block_offset, n):
    m = rows.shape[0]

    def body(x_ref, ids_ref, g_ref, b_ref, *rest):
        o_ref = rest[-1]
        x = x_ref[...]
        mean = jnp.mean(x, axis=1, keepdims=True)
        xc = x - mean
        var = jnp.mean(xc * xc, axis=1, keepdims=True)
        normed = xc * lax.rsqrt(var + EPS)
        out = normed * g_ref[...] + b_ref[...]
        pad = ids_ref[...] == PAD_IDX
        o_ref[...] = jnp.where(pad, b_ref[...], out)

    in_specs = [
        pl.BlockSpec((_LN_ROWS, HIDDEN), lambda i: (i, 0)),
        pl.BlockSpec((_LN_ROWS, 1), lambda i: (i, 0)),
        pl.BlockSpec((1, HIDDEN), lambda i: (0, 0)),
        pl.BlockSpec((1, HIDDEN), lambda i: (0, 0)),
    ]
    args = [rows, ids_col, gamma_row, beta_row]
    aliases = {}
    if big is not None:
        in_specs.append(pl.BlockSpec(memory_space=pl.ANY))
        args.append(big)
        aliases = {4: 0}

    return pl.pallas_call(
        body,
        grid=(m // _LN_ROWS,),
        in_specs=in_specs,
        out_specs=pl.BlockSpec(
            (_LN_ROWS, HIDDEN), lambda i, o=block_offset: (o + i, 0)
        ),
        out_shape=jax.ShapeDtypeStruct((n, HIDDEN), jnp.float32),
        input_output_aliases=aliases,
    )(*args)


def kernel(input_ids, table, gamma, beta):
    b, s = input_ids.shape
    ids = input_ids.reshape(-1).astype(jnp.int32)
    n = ids.shape[0]
    m = n // _K_SLICES
    gamma_row = gamma.reshape(1, HIDDEN)
    beta_row = beta.reshape(1, HIDDEN)

    big = None
    for k in range(_K_SLICES):
        ids_k = lax.slice(ids, (k * m,), ((k + 1) * m,))
        rows_k = _sc_gather(table, ids_k)
        big = _tc_layernorm_slice(
            rows_k, ids_k.reshape(m, 1), gamma_row, beta_row,
            big, k * (m // _LN_ROWS), n,
        )
    return big.reshape(b, s, HIDDEN)


# speedup vs baseline: 1.0240x; 1.0240x over previous
import functools

import jax
import jax.numpy as jnp
from jax import lax
from jax.experimental import pallas as pl
from jax.experimental.pallas import tpu as pltpu
from jax.experimental.pallas import tpu_sc as plsc

HIDDEN = 768
EPS = 1e-5
PAD_IDX = 0

_NUM_CORES = 2
_NUM_SUBCORES = 16

_CHUNK = 64

_LN_ROWS = 1024


def _sc_gather(table, ids):
    n = ids.shape[0]
    n_tiles = _NUM_CORES * _NUM_SUBCORES
    rows_per_tile = n // n_tiles
    n_chunks = rows_per_tile // _CHUNK
    mesh = plsc.VectorSubcoreMesh(core_axis_name="core", subcore_axis_name="subcore")

    @functools.partial(
        pl.kernel,
        out_type=jax.ShapeDtypeStruct((n, HIDDEN), jnp.float32),
        mesh=mesh,
        scratch_types=[
            pltpu.VMEM((rows_per_tile,), jnp.int32),
            pltpu.VMEM((_CHUNK, HIDDEN), jnp.float32),
            pltpu.VMEM((_CHUNK, HIDDEN), jnp.float32),
            pltpu.SemaphoreType.DMA,
            pltpu.SemaphoreType.DMA,
            pltpu.SemaphoreType.DMA,
            pltpu.SemaphoreType.DMA,
        ],
    )
    def gather_kernel(table_hbm, idx_hbm, out_hbm,
                      idx_v, buf0, buf1, g0, g1, s0, s1):
        wid = lax.axis_index("subcore") * _NUM_CORES + lax.axis_index("core")
        base = wid * rows_per_tile
        pltpu.sync_copy(idx_hbm.at[pl.ds(base, rows_per_tile)], idx_v)

        def gather_copy(c, buf, sem):
            return pltpu.make_async_copy(
                table_hbm.at[idx_v.at[pl.ds(c * _CHUNK, _CHUNK)]], buf, sem
            )

        def store_copy(c, buf, sem):
            return pltpu.make_async_copy(
                buf, out_hbm.at[pl.ds(base + c * _CHUNK, _CHUNK)], sem
            )

        gather_copy(0, buf0, g0).start()

        @pl.loop(0, n_chunks, step=2)
        def _(c):
            @pl.when(c + 1 < n_chunks)
            def _():
                gather_copy(c + 1, buf1, g1).start()

            gather_copy(c, buf0, g0).wait()
            store_copy(c, buf0, s0).start()
            store_copy(c, buf0, s0).wait()

            @pl.when(c + 2 < n_chunks)
            def _():
                gather_copy(c + 2, buf0, g0).start()

            @pl.when(c + 1 < n_chunks)
            def _():
                gather_copy(c + 1, buf1, g1).wait()
                store_copy(c + 1, buf1, s1).start()
                store_copy(c + 1, buf1, s1).wait()

    return gather_kernel(table, ids)


_K_SLICES = 2


def _tc_layernorm_slice(rows, ids_col, gamma_row, beta_row, big, block_offset, n):
    m = rows.shape[0]

    def body(x_ref, ids_ref, g_ref, b_ref, *rest):
        o_ref = rest[-1]
        x = x_ref[...]
        mean = jnp.mean(x, axis=1, keepdims=True)
        xc = x - mean
        var = jnp.mean(xc * xc, axis=1, keepdims=True)
        normed = xc * lax.rsqrt(var + EPS)
        out = normed * g_ref[...] + b_ref[...]
        pad = ids_ref[...] == PAD_IDX
        o_ref[...] = jnp.where(pad, b_ref[...], out)

    in_specs = [
        pl.BlockSpec((_LN_ROWS, HIDDEN), lambda i: (i, 0)),
        pl.BlockSpec((_LN_ROWS, 1), lambda i: (i, 0)),
        pl.BlockSpec((1, HIDDEN), lambda i: (0, 0)),
        pl.BlockSpec((1, HIDDEN), lambda i: (0, 0)),
    ]
    args = [rows, ids_col, gamma_row, beta_row]
    aliases = {}
    if big is not None:
        in_specs.append(pl.BlockSpec(memory_space=pl.ANY))
        args.append(big)
        aliases = {4: 0}

    return pl.pallas_call(
        body,
        grid=(m // _LN_ROWS,),
        in_specs=in_specs,
        out_specs=pl.BlockSpec(
            (_LN_ROWS, HIDDEN), lambda i, o=block_offset: (o + i, 0)
        ),
        out_shape=jax.ShapeDtypeStruct((n, HIDDEN), jnp.float32),
        input_output_aliases=aliases,
    )(*args)


def kernel(input_ids, table, gamma, beta):
    b, s = input_ids.shape
    ids = input_ids.reshape(-1).astype(jnp.int32)
    n = ids.shape[0]
    m = n // _K_SLICES
    gamma_row = gamma.reshape(1, HIDDEN)
    beta_row = beta.reshape(1, HIDDEN)

    big = None
    for k in range(_K_SLICES):
        ids_k = lax.slice(ids, (k * m,), ((k + 1) * m,))
        rows_k = _sc_gather(table, ids_k)
        big = _tc_layernorm_slice(
            rows_k, ids_k.reshape(m, 1), gamma_row, beta_row,
            big, k * (m // _LN_ROWS), n,
        )
    return big.reshape(b, s, HIDDEN)
